# Initial kernel scaffold; baseline (speedup 1.0000x reference)
import jax
import jax.numpy as jnp
from jax.experimental import pallas as pl


def kernel(edge, p_table, f_table, fc_w, fc_b):
    raise NotImplementedError("write your pallas kernel here")


# speedup vs baseline: 1.5432x; 1.5432x over previous
import functools

import jax
import jax.numpy as jnp
from jax import lax
from jax.experimental import pallas as pl
from jax.experimental.pallas import tpu as pltpu
from jax.experimental.pallas import tpu_sc as plsc

_N_F = 7
_N_P = 16
_COLS = _N_F + _N_P
_D = 64
_CHUNK = 512
_LANES = 16
_GROUPS = _CHUNK // _LANES
_N_TILES = 32


def _table_body(f_ref, p_ref, wf_ref, wp_ref, b_ref, t_ref):
    wf = jnp.dot(f_ref[...], wf_ref[...], preferred_element_type=jnp.float32)
    wp = jnp.dot(p_ref[...], wp_ref[...], preferred_element_type=jnp.float32)
    t_ref[...] = wf[:, None, :] + wp[None, :, :] + b_ref[...]


def _build_table(f_table, p_table, fc_w, fc_b):
    t3 = pl.pallas_call(
        _table_body,
        out_shape=jax.ShapeDtypeStruct((_N_F, _N_P, _D), jnp.float32),
    )(
        f_table,
        p_table,
        fc_w[:, :_N_P].T,
        fc_w[:, _N_P:].T,
        fc_b.reshape(1, 1, _D),
    )
    return t3.reshape(_N_F * _N_P, _D)


def _edge_embed(edge_flat, t_flat, n):
    assert n % _CHUNK == 0
    n_chunks = n // _CHUNK
    mesh = plsc.VectorSubcoreMesh(core_axis_name="c", subcore_axis_name="s")

    @functools.partial(
        pl.kernel,
        mesh=mesh,
        compiler_params=pltpu.CompilerParams(needs_layout_passes=False),
        out_type=jax.ShapeDtypeStruct((n * _D,), jnp.float32),
        scratch_types=[
            pltpu.VMEM((_CHUNK * _COLS,), jnp.float32),
            pltpu.VMEM((_N_F * _N_P * _D,), jnp.float32),
            pltpu.VMEM((_CHUNK * _D,), jnp.float32),
        ],
    )
    def run(edge_hbm, t_hbm, out_hbm, edge_v, t_v, out_v):
        cid = lax.axis_index("c")
        sid = lax.axis_index("s")
        wid = sid * 2 + cid
        pltpu.sync_copy(t_hbm, t_v)
        my_n = (n_chunks - wid + _N_TILES - 1) // _N_TILES

        def chunk_body(it, carry):
            chunk = wid + it * _N_TILES
            pltpu.sync_copy(
                edge_hbm.at[pl.ds(chunk * _CHUNK * _COLS, _CHUNK * _COLS)], edge_v)

            def group_body(g, c2):
                rows = g * _LANES + lax.iota(jnp.int32, _LANES)
                rb = rows * _COLS
                fmax = plsc.load_gather(edge_v, [rb])
                fidx = jnp.zeros((_LANES,), jnp.int32)
                for c in range(1, _N_F):
                    v = plsc.load_gather(edge_v, [rb + c])
                    gt = v > fmax
                    fidx = jnp.where(gt, jnp.full((_LANES,), c, jnp.int32), fidx)
                    fmax = jnp.maximum(fmax, v)
                pmax = plsc.load_gather(edge_v, [rb + _N_F])
                pidx = jnp.zeros((_LANES,), jnp.int32)
                for c in range(1, _N_P):
                    v = plsc.load_gather(edge_v, [rb + (_N_F + c)])
                    gt = v > pmax
                    pidx = jnp.where(gt, jnp.full((_LANES,), c, jnp.int32), pidx)
                    pmax = jnp.maximum(pmax, v)
                tb = (fidx * _N_P + pidx) * _D
                ob = rows * _D
                for d in range(_D):
                    col = plsc.load_gather(t_v, [tb + d])
                    plsc.store_scatter(out_v, [ob + d], col)
                return c2

            lax.fori_loop(0, _GROUPS, group_body, 0)
            pltpu.sync_copy(
                out_v, out_hbm.at[pl.ds(chunk * _CHUNK * _D, _CHUNK * _D)])
            return carry

        lax.fori_loop(0, my_n, chunk_body, 0)

    return run(edge_flat, t_flat)


def kernel(edge, p_table, f_table, fc_w, fc_b):
    n = edge.shape[0]
    t = _build_table(f_table, p_table, fc_w, fc_b)
    out_flat = _edge_embed(edge.reshape(-1), t.reshape(-1), n)
    return out_flat.reshape(n, _D)


# speedup vs baseline: 3.1787x; 2.0598x over previous
import functools

import jax
import jax.numpy as jnp
from jax import lax
from jax.experimental import pallas as pl
from jax.experimental.pallas import tpu as pltpu
from jax.experimental.pallas import tpu_sc as plsc

_N_F = 7
_N_P = 16
_COLS = _N_F + _N_P
_D = 64
_CHUNK = 512
_LANES = 16
_GROUPS = _CHUNK // _LANES
_N_TILES = 32


def _table_body(f_ref, p_ref, wf_ref, wp_ref, b_ref, t_ref):
    wf = jnp.dot(f_ref[...], wf_ref[...], preferred_element_type=jnp.float32)
    wp = jnp.dot(p_ref[...], wp_ref[...], preferred_element_type=jnp.float32)
    t_ref[...] = wf[:, None, :] + wp[None, :, :] + b_ref[...]


def _build_table(f_table, p_table, fc_w, fc_b):
    t3 = pl.pallas_call(
        _table_body,
        out_shape=jax.ShapeDtypeStruct((_N_F, _N_P, _D), jnp.float32),
    )(
        f_table,
        p_table,
        fc_w[:, :_N_P].T,
        fc_w[:, _N_P:].T,
        fc_b.reshape(1, 1, _D),
    )
    return t3.reshape(_N_F * _N_P, _D)


def _edge_embed(edge_flat, t_flat, n):
    assert n % _CHUNK == 0
    n_chunks = n // _CHUNK
    mesh = plsc.VectorSubcoreMesh(core_axis_name="c", subcore_axis_name="s")

    @functools.partial(
        pl.kernel,
        mesh=mesh,
        compiler_params=pltpu.CompilerParams(needs_layout_passes=False),
        out_type=jax.ShapeDtypeStruct((n * _D,), jnp.float32),
        scratch_types=[
            pltpu.VMEM((_CHUNK * _COLS,), jnp.float32),
            pltpu.VMEM((_N_F * _N_P * _D,), jnp.float32),
            pltpu.VMEM((_CHUNK * _D,), jnp.float32),
        ],
    )
    def run(edge_hbm, t_hbm, out_hbm, edge_v, t_v, out_v):
        cid = lax.axis_index("c")
        sid = lax.axis_index("s")
        wid = sid * 2 + cid
        pltpu.sync_copy(t_hbm, t_v)
        my_n = (n_chunks - wid + _N_TILES - 1) // _N_TILES

        def chunk_body(it, carry):
            chunk = wid + it * _N_TILES
            pltpu.sync_copy(
                edge_hbm.at[pl.ds(chunk * _CHUNK * _COLS, _CHUNK * _COLS)], edge_v)

            def group_body(g, c2):
                rows = g * _LANES + lax.iota(jnp.int32, _LANES)
                rb = rows * _COLS
                fmax = plsc.load_gather(edge_v, [rb])
                fidx = jnp.zeros((_LANES,), jnp.int32)
                for c in range(1, _N_F):
                    v = plsc.load_gather(edge_v, [rb + c])
                    gt = v > fmax
                    fidx = jnp.where(gt, jnp.full((_LANES,), c, jnp.int32), fidx)
                    fmax = jnp.maximum(fmax, v)
                pmax = plsc.load_gather(edge_v, [rb + _N_F])
                pidx = jnp.zeros((_LANES,), jnp.int32)
                for c in range(1, _N_P):
                    v = plsc.load_gather(edge_v, [rb + (_N_F + c)])
                    gt = v > pmax
                    pidx = jnp.where(gt, jnp.full((_LANES,), c, jnp.int32), pidx)
                    pmax = jnp.maximum(pmax, v)
                comb = (fidx * _N_P + pidx) * _D
                rb0 = g * (_LANES * _D)
                for i in range(_LANES):
                    tb = comb[i]
                    ob = rb0 + i * _D
                    for q in range(0, _D, _LANES):
                        out_v[pl.ds(ob + q, _LANES)] = t_v[pl.ds(tb + q, _LANES)]
                return c2

            lax.fori_loop(0, _GROUPS, group_body, 0)
            pltpu.sync_copy(
                out_v, out_hbm.at[pl.ds(chunk * _CHUNK * _D, _CHUNK * _D)])
            return carry

        lax.fori_loop(0, my_n, chunk_body, 0)

    return run(edge_flat, t_flat)


def kernel(edge, p_table, f_table, fc_w, fc_b):
    n = edge.shape[0]
    t = _build_table(f_table, p_table, fc_w, fc_b)
    out_flat = _edge_embed(edge.reshape(-1), t.reshape(-1), n)
    return out_flat.reshape(n, _D)


# speedup vs baseline: 3.4929x; 1.0988x over previous
import functools

import jax
import jax.numpy as jnp
from jax import lax
from jax.experimental import pallas as pl
from jax.experimental.pallas import tpu as pltpu
from jax.experimental.pallas import tpu_sc as plsc

_N_F = 7
_N_P = 16
_COLS = _N_F + _N_P
_D = 64
_CHUNK = 512
_LANES = 16
_GROUPS = _CHUNK // _LANES
_N_TILES = 32


def _table_body(f_ref, p_ref, wf_ref, wp_ref, b_ref, t_ref):
    wf = jnp.dot(f_ref[...], wf_ref[...], preferred_element_type=jnp.float32)
    wp = jnp.dot(p_ref[...], wp_ref[...], preferred_element_type=jnp.float32)
    t_ref[...] = wf[:, None, :] + wp[None, :, :] + b_ref[...]


def _build_table(f_table, p_table, fc_w, fc_b):
    t3 = pl.pallas_call(
        _table_body,
        out_shape=jax.ShapeDtypeStruct((_N_F, _N_P, _D), jnp.float32),
    )(
        f_table,
        p_table,
        fc_w[:, :_N_P].T,
        fc_w[:, _N_P:].T,
        fc_b.reshape(1, 1, _D),
    )
    return t3.reshape(_N_F * _N_P, _D)


def _edge_embed(edge_flat, t_flat, n):
    assert n % _CHUNK == 0
    n_chunks = n // _CHUNK
    mesh = plsc.VectorSubcoreMesh(core_axis_name="c", subcore_axis_name="s")

    @functools.partial(
        pl.kernel,
        mesh=mesh,
        compiler_params=pltpu.CompilerParams(needs_layout_passes=False),
        out_type=jax.ShapeDtypeStruct((n * _D,), jnp.float32),
        scratch_types=[
            pltpu.VMEM((_CHUNK * _COLS,), jnp.float32),
            pltpu.VMEM((_CHUNK * _COLS,), jnp.float32),
            pltpu.VMEM((_CHUNK * _D,), jnp.float32),
            pltpu.VMEM((_CHUNK * _D,), jnp.float32),
            pltpu.VMEM((_N_F * _N_P * _D,), jnp.float32),
            pltpu.SemaphoreType.DMA,
            pltpu.SemaphoreType.DMA,
            pltpu.SemaphoreType.DMA,
            pltpu.SemaphoreType.DMA,
        ],
    )
    def run(edge_hbm, t_hbm, out_hbm, edge_v0, edge_v1, out_v0, out_v1, t_v,
            si0, si1, so0, so1):
        cid = lax.axis_index("c")
        sid = lax.axis_index("s")
        wid = sid * 2 + cid
        pltpu.sync_copy(t_hbm, t_v)
        my_n = (n_chunks - wid + _N_TILES - 1) // _N_TILES
        eb = _CHUNK * _COLS
        ob = _CHUNK * _D
        edge_bufs = (edge_v0, edge_v1)
        out_bufs = (out_v0, out_v1)
        sin = (si0, si1)
        sout = (so0, so1)

        def in_slice(i):
            return edge_hbm.at[pl.ds((wid + i * _N_TILES) * eb, eb)]

        def out_slice(i):
            return out_hbm.at[pl.ds((wid + i * _N_TILES) * ob, ob)]

        def compute(edge_v, out_v):
            def group_body(g, c2):
                rows = g * _LANES + lax.iota(jnp.int32, _LANES)
                rb = rows * _COLS
                fmax = plsc.load_gather(edge_v, [rb])
                fidx = jnp.zeros((_LANES,), jnp.int32)
                for c in range(1, _N_F):
                    v = plsc.load_gather(edge_v, [rb + c])
                    gt = v > fmax
                    fidx = jnp.where(gt, jnp.full((_LANES,), c, jnp.int32), fidx)
                    fmax = jnp.maximum(fmax, v)
                pmax = plsc.load_gather(edge_v, [rb + _N_F])
                pidx = jnp.zeros((_LANES,), jnp.int32)
                for c in range(1, _N_P):
                    v = plsc.load_gather(edge_v, [rb + (_N_F + c)])
                    gt = v > pmax
                    pidx = jnp.where(gt, jnp.full((_LANES,), c, jnp.int32), pidx)
                    pmax = jnp.maximum(pmax, v)
                comb = (fidx * _N_P + pidx) * _D
                rb0 = g * (_LANES * _D)
                for i in range(_LANES):
                    tb = comb[i]
                    ob = rb0 + i * _D
                    for q in range(0, _D, _LANES):
                        out_v[pl.ds(ob + q, _LANES)] = t_v[pl.ds(tb + q, _LANES)]
                return c2

            lax.fori_loop(0, _GROUPS, group_body, 0)

        for b in range(2):
            @pl.when(b < my_n)
            def _():
                pltpu.async_copy(in_slice(b), edge_bufs[b], sin[b])

        def outer(j, carry):
            for b in range(2):
                i = 2 * j + b

                @pl.when(i < my_n)
                def _():
                    pltpu.make_async_copy(in_slice(i), edge_bufs[b], sin[b]).wait()

                    @pl.when(i >= 2)
                    def _():
                        pltpu.make_async_copy(
                            out_bufs[b], out_slice(i - 2), sout[b]).wait()

                    compute(edge_bufs[b], out_bufs[b])
                    pltpu.async_copy(out_bufs[b], out_slice(i), sout[b])

                    @pl.when(i + 2 < my_n)
                    def _():
                        pltpu.async_copy(in_slice(i + 2), edge_bufs[b], sin[b])
            return carry

        lax.fori_loop(0, (my_n + 1) // 2, outer, 0)

        for b in range(2):
            @pl.when(b < my_n)
            def _():
                pltpu.make_async_copy(out_bufs[b], out_slice(b), sout[b]).wait()

    return run(edge_flat, t_flat)


def kernel(edge, p_table, f_table, fc_w, fc_b):
    n = edge.shape[0]
    t = _build_table(f_table, p_table, fc_w, fc_b)
    out_flat = _edge_embed(edge.reshape(-1), t.reshape(-1), n)
    return out_flat.reshape(n, _D)
